# baseline (speedup 1.0000x reference)
import functools

import jax
import jax.numpy as jnp
from jax.experimental import pallas as pl
from jax.experimental.pallas import tpu as pltpu

INPUT_DIM = 2048
NUM_EXPERTS = 64
NUM_TOKENS = 16384
TILE = 2048
QTILE = TILE // 4


def _top2(x, gn):
    ss = jnp.sum(x * x, axis=1, keepdims=True)
    xn = x / jnp.maximum(jnp.sqrt(ss), 1e-12)
    logits = jax.lax.dot_general(
        xn, gn, (((1,), (1,)), ((), ())),
        preferred_element_type=jnp.float32)
    lt = logits.T
    idx = jax.lax.broadcasted_iota(jnp.int32, lt.shape, 0)
    m1 = jnp.max(lt, axis=0, keepdims=True)
    i1 = jnp.min(jnp.where(lt == m1, idx, NUM_EXPERTS), axis=0,
                 keepdims=True)
    masked = jnp.where(idx == i1, -jnp.inf, lt)
    m2 = jnp.max(masked, axis=0, keepdims=True)
    i2 = jnp.min(jnp.where(masked == m2, idx, NUM_EXPERTS), axis=0,
                 keepdims=True)
    e2 = jnp.exp(m2 - m1)
    w1 = 1.0 / (1.0 + e2)
    w2 = e2 * w1
    return (jnp.concatenate([w1, w2], axis=0),
            jnp.concatenate([i1, i2], axis=0))


def _router_body(xa_ref, xb_ref, xc_ref, xd_ref, g_ref, w_ref, i_ref, gn_ref):
    @pl.when(pl.program_id(0) == 0)
    def _():
        g = g_ref[...]
        gss = jnp.sum(g * g, axis=1, keepdims=True)
        gn_ref[...] = g / jnp.maximum(jnp.sqrt(gss), 1e-12)

    gn = gn_ref[...]
    wa, ia = _top2(xa_ref[...], gn)
    wb, ib = _top2(xb_ref[...], gn)
    wc, ic = _top2(xc_ref[...], gn)
    wd, id_ = _top2(xd_ref[...], gn)
    w_ref[...] = jnp.concatenate([wa, wb, wc, wd], axis=1)
    i_ref[...] = jnp.concatenate([ia, ib, ic, id_], axis=1)


@functools.partial(jax.jit, static_argnames=("interpret",))
def kernel(x, latent_genomes, interpret=False):
    n_tiles = NUM_TOKENS // TILE
    weights_t, indices_t = pl.pallas_call(
        _router_body,
        grid=(n_tiles,),
        in_specs=[
            pl.BlockSpec((QTILE, INPUT_DIM), lambda i: (4 * i, 0)),
            pl.BlockSpec((QTILE, INPUT_DIM), lambda i: (4 * i + 1, 0)),
            pl.BlockSpec((QTILE, INPUT_DIM), lambda i: (4 * i + 2, 0)),
            pl.BlockSpec((QTILE, INPUT_DIM), lambda i: (4 * i + 3, 0)),
            pl.BlockSpec((NUM_EXPERTS, INPUT_DIM), lambda i: (0, 0)),
        ],
        out_specs=[
            pl.BlockSpec((2, TILE), lambda i: (0, i)),
            pl.BlockSpec((2, TILE), lambda i: (0, i)),
        ],
        out_shape=[
            jax.ShapeDtypeStruct((2, NUM_TOKENS), jnp.float32),
            jax.ShapeDtypeStruct((2, NUM_TOKENS), jnp.int32),
        ],
        scratch_shapes=[pltpu.VMEM((NUM_EXPERTS, INPUT_DIM), jnp.float32)],
        compiler_params=pltpu.CompilerParams(
            dimension_semantics=("arbitrary",)),
        interpret=interpret,
    )(x, x, x, x, latent_genomes)
    return (weights_t.T, indices_t.T)


# speedup vs baseline: 1.0065x; 1.0065x over previous
import functools

import jax
import jax.numpy as jnp
from jax.experimental import pallas as pl
from jax.experimental.pallas import tpu as pltpu

INPUT_DIM = 2048
NUM_EXPERTS = 64
NUM_TOKENS = 16384
TILE = 2048
HTILE = TILE // 2


def _top2(x, gn):
    ss = jnp.sum(x * x, axis=1, keepdims=True)
    xn = x / jnp.maximum(jnp.sqrt(ss), 1e-12)
    logits = jax.lax.dot_general(
        xn, gn, (((1,), (1,)), ((), ())),
        preferred_element_type=jnp.float32)
    lt = logits.T
    idx = jax.lax.broadcasted_iota(jnp.int32, lt.shape, 0)
    m1 = jnp.max(lt, axis=0, keepdims=True)
    i1 = jnp.min(jnp.where(lt == m1, idx, NUM_EXPERTS), axis=0,
                 keepdims=True)
    masked = jnp.where(idx == i1, -jnp.inf, lt)
    m2 = jnp.max(masked, axis=0, keepdims=True)
    i2 = jnp.min(jnp.where(masked == m2, idx, NUM_EXPERTS), axis=0,
                 keepdims=True)
    e2 = jnp.exp(m2 - m1)
    w1 = 1.0 / (1.0 + e2)
    w2 = e2 * w1
    return (jnp.concatenate([w1, w2], axis=0),
            jnp.concatenate([i1, i2], axis=0))


def _router_body(xa_ref, xb_ref, g_ref, w_ref, i_ref, gn_ref):
    @pl.when(pl.program_id(0) == 0)
    def _():
        g = g_ref[...]
        gss = jnp.sum(g * g, axis=1, keepdims=True)
        gn_ref[...] = g / jnp.maximum(jnp.sqrt(gss), 1e-12)

    gn = gn_ref[...]
    wa, ia = _top2(xa_ref[...], gn)
    wb, ib = _top2(xb_ref[...], gn)
    w_ref[...] = jnp.concatenate([wa, wb], axis=1)
    i_ref[...] = jnp.concatenate([ia, ib], axis=1)


@functools.partial(jax.jit, static_argnames=("interpret",))
def kernel(x, latent_genomes, interpret=False):
    n_tiles = NUM_TOKENS // TILE
    weights_t, indices_t = pl.pallas_call(
        _router_body,
        grid=(n_tiles,),
        in_specs=[
            pl.BlockSpec((HTILE, INPUT_DIM), lambda i: (2 * i, 0)),
            pl.BlockSpec((HTILE, INPUT_DIM), lambda i: (2 * i + 1, 0)),
            pl.BlockSpec((NUM_EXPERTS, INPUT_DIM), lambda i: (0, 0)),
        ],
        out_specs=[
            pl.BlockSpec((2, TILE), lambda i: (0, i)),
            pl.BlockSpec((2, TILE), lambda i: (0, i)),
        ],
        out_shape=[
            jax.ShapeDtypeStruct((2, NUM_TOKENS), jnp.float32),
            jax.ShapeDtypeStruct((2, NUM_TOKENS), jnp.int32),
        ],
        scratch_shapes=[pltpu.VMEM((NUM_EXPERTS, INPUT_DIM), jnp.float32)],
        compiler_params=pltpu.CompilerParams(
            dimension_semantics=("arbitrary",)),
        interpret=interpret,
    )(x, x, latent_genomes)
    return (weights_t.T, indices_t.T)
